# baseline (speedup 1.0000x reference)
import functools
import math

import jax
import jax.numpy as jnp
from jax import lax
from jax.experimental import pallas as pl
from jax.experimental.pallas import tpu as pltpu
from jax.experimental.pallas import tpu_sc as plsc

_NC = 2
_NS = 16
_NW = _NC * _NS
_CHUNK = 128

_NBUF = 3
_NIDX = 4
_PREG = 2
_PREI = 3
_LAG = _NBUF - _PREG
assert _LAG == _NIDX - _PREI
_U = _NBUF * _NIDX // math.gcd(_NBUF, _NIDX)


@functools.lru_cache(maxsize=None)
def _make_agg(n, e, d, interpret=False):
    n_chunks = e // _CHUNK
    kbase, kext = divmod(n_chunks, _NW)
    slab = (n // _NS) // 8 * 8
    rem = n - slab * _NS
    mesh = plsc.VectorSubcoreMesh(core_axis_name="c", subcore_axis_name="s",
                                  num_cores=_NC, num_subcores=_NS)

    @functools.partial(
        pl.kernel,
        out_type=jax.ShapeDtypeStruct((2 * n, d), jnp.float32),
        mesh=mesh,
        scratch_types=[
            pltpu.VMEM((_NIDX, _CHUNK), jnp.int32),
            pltpu.VMEM((_NIDX, _CHUNK), jnp.int32),
            pltpu.VMEM((_NBUF, _CHUNK, d), jnp.float32),
            pltpu.VMEM_SHARED((n, d), jnp.float32),
            [pltpu.SemaphoreType.DMA] * _NIDX,
            [pltpu.SemaphoreType.DMA] * _NBUF,
            [pltpu.SemaphoreType.DMA] * _NBUF,
            pltpu.SemaphoreType.DMA,
        ],
        interpret=interpret,
    )
    def agg(h_hbm, src_hbm, dst_hbm, out_hbm, sidx, didx, bufs, acc_sh,
            isems, gsems, ssems, zsem):
        cid = lax.axis_index("c")
        sid = lax.axis_index("s")
        wid = sid * _NC + cid
        r0 = sid * slab
        k = kbase + jnp.where(wid < kext, 1, 0)
        e0 = (wid * kbase + jnp.minimum(wid, kext)) * _CHUNK

        def idx_start(j, q):
            pltpu.async_copy(src_hbm.at[pl.ds(e0 + j * _CHUNK, _CHUNK)],
                             sidx.at[q], isems[q])
            pltpu.async_copy(dst_hbm.at[pl.ds(e0 + j * _CHUNK, _CHUNK)],
                             didx.at[q], isems[q])

        def idx_wait(j, q):
            pltpu.make_async_copy(src_hbm.at[pl.ds(e0 + j * _CHUNK, _CHUNK)],
                                  sidx.at[q], isems[q]).wait()
            pltpu.make_async_copy(dst_hbm.at[pl.ds(e0 + j * _CHUNK, _CHUNK)],
                                  didx.at[q], isems[q]).wait()

        def gather_start(q, s):
            pltpu.async_copy(h_hbm.at[sidx.at[q]], bufs.at[s], gsems[s])

        def gather_wait(q, s):
            pltpu.make_async_copy(h_hbm.at[sidx.at[q]], bufs.at[s],
                                  gsems[s]).wait()

        def scatter_start(q, s):
            pltpu.async_copy(bufs.at[s], acc_sh.at[didx.at[q]], ssems[s],
                             add=True)

        def scatter_wait(q, s):
            pltpu.make_async_copy(bufs.at[s], acc_sh.at[didx.at[q]],
                                  ssems[s]).wait()

        pltpu.async_copy(h_hbm.at[pl.ds(r0, slab)], acc_sh.at[pl.ds(r0, slab)],
                         zsem)
        if rem:
            @pl.when(sid == _NS - 1)
            def _():
                pltpu.async_copy(h_hbm.at[pl.ds(_NS * slab, rem)],
                                 acc_sh.at[pl.ds(_NS * slab, rem)], zsem)

        for j in range(_PREI):
            @pl.when(j < k)
            def _(j=j):
                idx_start(j, j % _NIDX)
        for j in range(_PREG):
            @pl.when(j < k)
            def _(j=j):
                idx_wait(j, j % _NIDX)
                gather_start(j % _NIDX, j % _NBUF)

        pltpu.make_async_copy(h_hbm.at[pl.ds(r0, slab)],
                              acc_sh.at[pl.ds(r0, slab)], zsem).wait()
        if rem:
            @pl.when(sid == _NS - 1)
            def _():
                pltpu.make_async_copy(h_hbm.at[pl.ds(_NS * slab, rem)],
                                      acc_sh.at[pl.ds(_NS * slab, rem)],
                                      zsem).wait()
        plsc.subcore_barrier()

        def step(i, t):
            q = t % _NIDX
            s = t % _NBUF
            qi = (q + _PREI) % _NIDX
            qg = (q + _PREG) % _NIDX
            sg = (s + _PREG) % _NBUF

            @pl.when(i >= _LAG)
            def _():
                scatter_wait(qi, sg)

            @pl.when(i + _PREI < k)
            def _():
                idx_start(i + _PREI, qi)

            @pl.when(i + _PREG < k)
            def _():
                idx_wait(i + _PREG, qg)
                gather_start(qg, sg)
            gather_wait(q, s)
            scatter_start(q, s)

        def body(i, carry):
            for t in range(_U):
                @pl.when(lax.rem(i, _U) == t)
                def _():
                    step(i, t)
            return carry

        lax.fori_loop(0, k, body, 0)
        for u in range(_LAG):
            @pl.when(k > u)
            def _():
                j = k - 1 - u
                for t in range(_U):
                    @pl.when(lax.rem(j, _U) == t)
                    def _():
                        scatter_wait(t % _NIDX, t % _NBUF)

        plsc.subcore_barrier()
        pltpu.sync_copy(acc_sh.at[pl.ds(r0, slab)],
                        out_hbm.at[pl.ds(cid * n + r0, slab)])
        if rem:
            @pl.when(sid == _NS - 1)
            def _():
                pltpu.sync_copy(acc_sh.at[pl.ds(_NS * slab, rem)],
                                out_hbm.at[pl.ds(cid * n + _NS * slab, rem)])

    return agg


def _bn(y, g, b):
    mu = jnp.mean(y, axis=0, keepdims=True)
    var = jnp.mean((y - mu) ** 2, axis=0, keepdims=True)
    return g * (y - mu) / jnp.sqrt(var + 1e-5) + b


@functools.lru_cache(maxsize=None)
def _make_mlp(n, d, h, last, out_d, interpret=False):
    def body(h_ref, p_ref, w1_ref, b1_ref, g1_ref, t1_ref,
             w2_ref, b2_ref, g2_ref, t2_ref, eps_ref, *rest):
        if last:
            wc_ref, bc_ref, out_ref, logits_ref = rest
        else:
            (out_ref,) = rest
        z = ((eps_ref[0] - 1.0) * h_ref[...]
             + p_ref[pl.ds(0, n), :] + p_ref[pl.ds(n, n), :])
        y = jnp.dot(z.astype(jnp.bfloat16), w1_ref[...].astype(jnp.bfloat16),
                    preferred_element_type=jnp.float32) + b1_ref[...]
        y = _bn(y, g1_ref[...], t1_ref[...])
        y = jnp.maximum(y, 0.0)
        y = jnp.dot(y.astype(jnp.bfloat16), w2_ref[...].astype(jnp.bfloat16),
                    preferred_element_type=jnp.float32) + b2_ref[...]
        y = _bn(y, g2_ref[...], t2_ref[...])
        if last:
            out_ref[...] = y
            logits_ref[...] = (jnp.dot(y, wc_ref[...],
                                       preferred_element_type=jnp.float32)
                               + bc_ref[...])
        else:
            out_ref[...] = jnp.maximum(y, 0.0)

    n_in = 13 if last else 11
    in_specs = [pl.BlockSpec(memory_space=pltpu.VMEM)] * n_in
    in_specs[10] = pl.BlockSpec(memory_space=pltpu.SMEM)
    if last:
        out_shape = (jax.ShapeDtypeStruct((n, h), jnp.float32),
                     jax.ShapeDtypeStruct((n, out_d), jnp.float32))
        out_specs = (pl.BlockSpec(memory_space=pltpu.VMEM),
                     pl.BlockSpec(memory_space=pltpu.VMEM))
    else:
        out_shape = jax.ShapeDtypeStruct((n, h), jnp.float32)
        out_specs = pl.BlockSpec(memory_space=pltpu.VMEM)
    return pl.pallas_call(body, out_shape=out_shape, in_specs=in_specs,
                          out_specs=out_specs, interpret=interpret)


def kernel(x, edge_index, params):
    n, d_in = x.shape
    e = edge_index.shape[1]
    src = edge_index[0]
    dst = edge_index[1]
    num_layers = 3
    h = x
    for i in range(num_layers):
        hdim = params["W1_%d" % i].shape[1]
        partials = _make_agg(n, e, h.shape[1])(h, src, dst)
        last = i == num_layers - 1
        eps = jnp.reshape(params["eps_%d" % i], (1,)).astype(jnp.float32)
        args = [h, partials,
                params["W1_%d" % i], jnp.reshape(params["b1_%d" % i], (1, hdim)),
                jnp.reshape(params["g1_%d" % i], (1, hdim)),
                jnp.reshape(params["bt1_%d" % i], (1, hdim)),
                params["W2_%d" % i], jnp.reshape(params["b2_%d" % i], (1, hdim)),
                jnp.reshape(params["g2_%d" % i], (1, hdim)),
                jnp.reshape(params["bt2_%d" % i], (1, hdim)),
                eps]
        if last:
            out_d = params["Wc"].shape[1]
            args += [params["Wc"], jnp.reshape(params["bc"], (1, out_d))]
            h, logits = _make_mlp(n, h.shape[1], hdim, True, out_d)(*args)
        else:
            h = _make_mlp(n, h.shape[1], hdim, False, 0)(*args)
    return logits, h


# speedup vs baseline: 1.0472x; 1.0472x over previous
import functools
import math

import jax
import jax.numpy as jnp
from jax import lax
from jax.experimental import pallas as pl
from jax.experimental.pallas import tpu as pltpu
from jax.experimental.pallas import tpu_sc as plsc

_NC = 2
_NS = 16
_NW = _NC * _NS
_CHUNK = 128

_NBUF = 3
_NIDX = 4
_PREG = 2
_PREI = 3
_LAG = _NBUF - _PREG
assert _LAG == _NIDX - _PREI
_U = _NBUF * _NIDX // math.gcd(_NBUF, _NIDX)


@functools.lru_cache(maxsize=None)
def _make_agg(n, e, d, interpret=False):
    n_chunks = e // _CHUNK
    kbase, kext = divmod(n_chunks, _NW)
    slab = (n // _NS) // 8 * 8
    rem = n - slab * _NS
    mesh = plsc.VectorSubcoreMesh(core_axis_name="c", subcore_axis_name="s",
                                  num_cores=_NC, num_subcores=_NS)

    @functools.partial(
        pl.kernel,
        out_type=jax.ShapeDtypeStruct((2 * n, d), jnp.float32),
        mesh=mesh,
        scratch_types=[
            pltpu.VMEM((_NIDX, _CHUNK), jnp.int32),
            pltpu.VMEM((_NIDX, _CHUNK), jnp.int32),
            pltpu.VMEM((_NBUF, _CHUNK, d), jnp.float32),
            pltpu.VMEM_SHARED((n, d), jnp.float32),
            [pltpu.SemaphoreType.DMA] * _NIDX,
            [pltpu.SemaphoreType.DMA] * _NBUF,
            [pltpu.SemaphoreType.DMA] * _NBUF,
            pltpu.SemaphoreType.DMA,
        ],
        interpret=interpret,
    )
    def agg(h_hbm, src_hbm, dst_hbm, out_hbm, sidx, didx, bufs, acc_sh,
            isems, gsems, ssems, zsem):
        cid = lax.axis_index("c")
        sid = lax.axis_index("s")
        wid = sid * _NC + cid
        r0 = sid * slab
        k = kbase + jnp.where(wid < kext, 1, 0)
        e0 = (wid * kbase + jnp.minimum(wid, kext)) * _CHUNK

        def idx_start(j, q):
            pltpu.async_copy(src_hbm.at[pl.ds(e0 + j * _CHUNK, _CHUNK)],
                             sidx.at[q], isems[q])
            pltpu.async_copy(dst_hbm.at[pl.ds(e0 + j * _CHUNK, _CHUNK)],
                             didx.at[q], isems[q])

        def idx_wait(j, q):
            pltpu.make_async_copy(src_hbm.at[pl.ds(e0 + j * _CHUNK, _CHUNK)],
                                  sidx.at[q], isems[q]).wait()
            pltpu.make_async_copy(dst_hbm.at[pl.ds(e0 + j * _CHUNK, _CHUNK)],
                                  didx.at[q], isems[q]).wait()

        def gather_start(q, s):
            pltpu.async_copy(h_hbm.at[sidx.at[q]], bufs.at[s], gsems[s])

        def gather_wait(q, s):
            pltpu.make_async_copy(h_hbm.at[sidx.at[q]], bufs.at[s],
                                  gsems[s]).wait()

        def scatter_start(q, s):
            pltpu.async_copy(bufs.at[s], acc_sh.at[didx.at[q]], ssems[s],
                             add=True)

        def scatter_wait(q, s):
            pltpu.make_async_copy(bufs.at[s], acc_sh.at[didx.at[q]],
                                  ssems[s]).wait()

        pltpu.async_copy(h_hbm.at[pl.ds(r0, slab)], acc_sh.at[pl.ds(r0, slab)],
                         zsem)
        if rem:
            @pl.when(sid == _NS - 1)
            def _():
                pltpu.async_copy(h_hbm.at[pl.ds(_NS * slab, rem)],
                                 acc_sh.at[pl.ds(_NS * slab, rem)], zsem)

        for j in range(_PREI):
            @pl.when(j < k)
            def _(j=j):
                idx_start(j, j % _NIDX)
        for j in range(_PREG):
            @pl.when(j < k)
            def _(j=j):
                idx_wait(j, j % _NIDX)
                gather_start(j % _NIDX, j % _NBUF)

        pltpu.make_async_copy(h_hbm.at[pl.ds(r0, slab)],
                              acc_sh.at[pl.ds(r0, slab)], zsem).wait()
        if rem:
            @pl.when(sid == _NS - 1)
            def _():
                pltpu.make_async_copy(h_hbm.at[pl.ds(_NS * slab, rem)],
                                      acc_sh.at[pl.ds(_NS * slab, rem)],
                                      zsem).wait()
        plsc.subcore_barrier()

        def step(i, t):
            q = t % _NIDX
            s = t % _NBUF
            qi = (q + _PREI) % _NIDX
            qg = (q + _PREG) % _NIDX
            sg = (s + _PREG) % _NBUF

            @pl.when(i >= _LAG)
            def _():
                scatter_wait(qi, sg)

            @pl.when(i + _PREI < k)
            def _():
                idx_start(i + _PREI, qi)

            @pl.when(i + _PREG < k)
            def _():
                idx_wait(i + _PREG, qg)
                gather_start(qg, sg)
            gather_wait(q, s)
            scatter_start(q, s)

        def body(i, carry):
            for t in range(_U):
                @pl.when(lax.rem(i, _U) == t)
                def _():
                    step(i, t)
            return carry

        lax.fori_loop(0, k, body, 0)
        for u in range(_LAG):
            @pl.when(k > u)
            def _():
                j = k - 1 - u
                for t in range(_U):
                    @pl.when(lax.rem(j, _U) == t)
                    def _():
                        scatter_wait(t % _NIDX, t % _NBUF)

        plsc.subcore_barrier()
        pltpu.sync_copy(acc_sh.at[pl.ds(r0, slab)],
                        out_hbm.at[pl.ds(cid * n + r0, slab)])
        if rem:
            @pl.when(sid == _NS - 1)
            def _():
                pltpu.sync_copy(acc_sh.at[pl.ds(_NS * slab, rem)],
                                out_hbm.at[pl.ds(cid * n + _NS * slab, rem)])

    return agg


def _bn(y, g, b, relu):
    inv_n = 1.0 / y.shape[0]
    mu = jnp.sum(y, axis=0, keepdims=True) * inv_n
    var = jnp.sum(y * y, axis=0, keepdims=True) * inv_n - mu * mu
    scale = g * lax.rsqrt(var + 1e-5)
    off = b - mu * scale
    out = y * scale + off
    return jnp.maximum(out, 0.0) if relu else out


@functools.lru_cache(maxsize=None)
def _make_mlp(n, d, h, last, out_d, interpret=False):
    def body(h_ref, p_ref, w1_ref, b1_ref, g1_ref, t1_ref,
             w2_ref, b2_ref, g2_ref, t2_ref, eps_ref, *rest):
        if last:
            wc_ref, bc_ref, out_ref, logits_ref = rest
        else:
            (out_ref,) = rest
        z = ((eps_ref[0] - 1.0) * h_ref[...]
             + p_ref[pl.ds(0, n), :] + p_ref[pl.ds(n, n), :])
        y = jnp.dot(z, w1_ref[...], preferred_element_type=jnp.float32) + b1_ref[...]
        y = _bn(y, g1_ref[...], t1_ref[...], True)
        y = jnp.dot(y, w2_ref[...], preferred_element_type=jnp.float32) + b2_ref[...]
        y = _bn(y, g2_ref[...], t2_ref[...], not last)
        if last:
            out_ref[...] = y
            logits_ref[...] = (jnp.dot(y, wc_ref[...],
                                       preferred_element_type=jnp.float32)
                               + bc_ref[...])
        else:
            out_ref[...] = y

    n_in = 13 if last else 11
    in_specs = [pl.BlockSpec(memory_space=pltpu.VMEM)] * n_in
    in_specs[10] = pl.BlockSpec(memory_space=pltpu.SMEM)
    if last:
        out_shape = (jax.ShapeDtypeStruct((n, h), jnp.float32),
                     jax.ShapeDtypeStruct((n, out_d), jnp.float32))
        out_specs = (pl.BlockSpec(memory_space=pltpu.VMEM),
                     pl.BlockSpec(memory_space=pltpu.VMEM))
    else:
        out_shape = jax.ShapeDtypeStruct((n, h), jnp.float32)
        out_specs = pl.BlockSpec(memory_space=pltpu.VMEM)
    return pl.pallas_call(body, out_shape=out_shape, in_specs=in_specs,
                          out_specs=out_specs, interpret=interpret)


def kernel(x, edge_index, params):
    n, d_in = x.shape
    e = edge_index.shape[1]
    src = edge_index[0]
    dst = edge_index[1]
    num_layers = 3
    h = x
    for i in range(num_layers):
        hdim = params["W1_%d" % i].shape[1]
        partials = _make_agg(n, e, h.shape[1])(h, src, dst)
        last = i == num_layers - 1
        eps = jnp.reshape(params["eps_%d" % i], (1,)).astype(jnp.float32)
        args = [h, partials,
                params["W1_%d" % i], jnp.reshape(params["b1_%d" % i], (1, hdim)),
                jnp.reshape(params["g1_%d" % i], (1, hdim)),
                jnp.reshape(params["bt1_%d" % i], (1, hdim)),
                params["W2_%d" % i], jnp.reshape(params["b2_%d" % i], (1, hdim)),
                jnp.reshape(params["g2_%d" % i], (1, hdim)),
                jnp.reshape(params["bt2_%d" % i], (1, hdim)),
                eps]
        if last:
            out_d = params["Wc"].shape[1]
            args += [params["Wc"], jnp.reshape(params["bc"], (1, out_d))]
            h, logits = _make_mlp(n, h.shape[1], hdim, True, out_d)(*args)
        else:
            h = _make_mlp(n, h.shape[1], hdim, False, 0)(*args)
    return logits, h
